# Initial kernel scaffold; baseline (speedup 1.0000x reference)
import jax
import jax.numpy as jnp
from jax.experimental import pallas as pl


def kernel(logits, val_freqs):
    raise NotImplementedError("write your pallas kernel here")


# speedup vs baseline: 362.9527x; 362.9527x over previous
import functools

import jax
import jax.numpy as jnp
import numpy as np
from jax import lax
from jax.experimental import pallas as pl
from jax.experimental.pallas import tpu as pltpu
from jax.experimental.pallas import tpu_sc as plsc

NUM_PROB_BINS = 15
NUM_CLASSES = 19
NUM_NEIGH = 9
B, H, W = 8, 512, 512
LANES = 16
CHUNKS = W // LANES
NUM_WORKERS = 32
WORKERS_PER_IMG = NUM_WORKERS // B
ROWS_PER_WORKER = H // WORKERS_PER_IMG
TABLE_N = NUM_CLASSES * NUM_NEIGH * NUM_PROB_BINS
TABLE_PAD = 2576
VROW = 544

_INV9 = np.float32(1.0) / np.float32(9.0)
_W9 = np.float32(1.0) / np.float32(NUM_NEIGH)
_W15 = np.float32(1.0) / np.float32(NUM_PROB_BINS)


def _body(logits_hbm, vf_hbm, out_hbm, table_v, prob_v, vbuf, obuf):
    cid = lax.axis_index("c")
    sid = lax.axis_index("s")
    wid = sid * 2 + cid
    b = wid // WORKERS_PER_IMG
    r0 = (wid % WORKERS_PER_IMG) * ROWS_PER_WORKER

    pltpu.sync_copy(vf_hbm, table_v)

    zero16 = jnp.zeros((LANES,), jnp.float32)

    def _zinit(i, carry):
        c = i // CHUNKS
        k = i % CHUNKS
        prob_v[4, c, pl.ds(k * LANES, LANES)] = zero16
        return carry

    lax.fori_loop(0, NUM_CLASSES * CHUNKS, _zinit, 0)

    for c in range(NUM_CLASSES):
        vbuf[pl.ds(c * VROW, LANES)] = zero16
        vbuf[pl.ds(c * VROW + LANES + W, LANES)] = zero16

    def compute_prob_row(h):
        slot = lax.rem(h, 4)
        pltpu.sync_copy(logits_hbm.at[b, :, h, :], prob_v.at[slot])

        def chunk(k, carry):
            o = k * LANES
            es = []
            s = zero16
            for c in range(NUM_CLASSES):
                e = jnp.exp(prob_v[slot, c, pl.ds(o, LANES)])
                es.append(e)
                s = s + e
            r = 1.0 / s
            for c in range(NUM_CLASSES):
                prob_v[slot, c, pl.ds(o, LANES)] = es[c] * r
            return carry

        lax.fori_loop(0, CHUNKS, chunk, 0)

    @pl.when(r0 > 0)
    def _():
        compute_prob_row(r0 - 1)

    compute_prob_row(r0)

    def row_body(h, carry):
        @pl.when(h < H - 1)
        def _():
            compute_prob_row(h + 1)

        s0 = lax.rem(h, 4)
        sm1 = jnp.where(h == 0, 4, lax.rem(h - 1, 4))
        sp1 = jnp.where(h < H - 1, lax.rem(h + 1, 4), 4)

        def vchunk(k, carry):
            o = k * LANES
            for c in range(NUM_CLASSES):
                v = (prob_v[sm1, c, pl.ds(o, LANES)]
                     + prob_v[s0, c, pl.ds(o, LANES)]
                     + prob_v[sp1, c, pl.ds(o, LANES)])
                vbuf[pl.ds(c * VROW + LANES + o, LANES)] = v
            return carry

        lax.fori_loop(0, CHUNKS, vchunk, 0)

        def hchunk(k, carry):
            o = k * LANES
            lanes = lax.iota(jnp.int32, LANES)
            cals = []
            s = zero16
            for c in range(NUM_CLASSES):
                base = lanes + (o + c * VROW + LANES)
                vm = plsc.load_gather(vbuf, [base - 1])
                vc = vbuf[pl.ds(c * VROW + LANES + o, LANES)]
                vp = plsc.load_gather(vbuf, [base + 1])
                hs = vm + vc + vp
                lc = hs * _INV9
                lb = jnp.minimum((lc / _W9).astype(jnp.int32), NUM_NEIGH - 1)
                p = prob_v[s0, c, pl.ds(o, LANES)]
                pb = jnp.minimum((p / _W15).astype(jnp.int32),
                                 NUM_PROB_BINS - 1)
                idx = lb * NUM_PROB_BINS + pb + c * (NUM_NEIGH * NUM_PROB_BINS)
                cal = plsc.load_gather(table_v, [idx])
                cals.append(cal)
                s = s + cal
            s = jnp.where(s == 0.0, 1.0, s)
            r = 1.0 / s
            for c in range(NUM_CLASSES):
                obuf[c, pl.ds(o, LANES)] = cals[c] * r
            return carry

        lax.fori_loop(0, CHUNKS, hchunk, 0)

        pltpu.sync_copy(obuf, out_hbm.at[b, :, h, :])
        return carry

    lax.fori_loop(r0, r0 + ROWS_PER_WORKER, row_body, 0)


def kernel(logits, val_freqs):
    vf_flat = jnp.pad(val_freqs.reshape(-1), (0, TABLE_PAD - TABLE_N))
    mesh = plsc.VectorSubcoreMesh(core_axis_name="c", subcore_axis_name="s")
    k = functools.partial(
        pl.kernel,
        mesh=mesh,
        out_type=jax.ShapeDtypeStruct((B, NUM_CLASSES, H, W), jnp.float32),
        scratch_types=[
            pltpu.VMEM((TABLE_PAD,), jnp.float32),
            pltpu.VMEM((5, NUM_CLASSES, W), jnp.float32),
            pltpu.VMEM((NUM_CLASSES * VROW,), jnp.float32),
            pltpu.VMEM((NUM_CLASSES, W), jnp.float32),
        ],
        compiler_params=pltpu.CompilerParams(needs_layout_passes=False),
    )(_body)
    return k(logits, vf_flat)


# speedup vs baseline: 385.9923x; 1.0635x over previous
import functools

import jax
import jax.numpy as jnp
import numpy as np
from jax import lax
from jax.experimental import pallas as pl
from jax.experimental.pallas import tpu as pltpu
from jax.experimental.pallas import tpu_sc as plsc

NUM_PROB_BINS = 15
NUM_CLASSES = 19
NUM_NEIGH = 9
B, H, W = 8, 512, 512
LANES = 16
CHUNKS = W // LANES
NUM_WORKERS = 32
WORKERS_PER_IMG = NUM_WORKERS // B
ROWS_PER_WORKER = H // WORKERS_PER_IMG
TABLE_N = NUM_CLASSES * NUM_NEIGH * NUM_PROB_BINS
TABLE_PAD = 2576
VROW = 544
QS = NUM_CLASSES * VROW

_WHI = np.float32(
    np.asarray(np.float32(1.0 / 9.0)).astype(__import__("ml_dtypes").bfloat16)
)
_W9 = np.float32(1.0) / np.float32(NUM_NEIGH)
_W15 = np.float32(1.0) / np.float32(NUM_PROB_BINS)


def _round_bf16(x):
    u = plsc.bitcast(x, jnp.int32)
    lsb = lax.shift_right_logical(u, 16) & 1
    r = (u + 32767 + lsb) & jnp.int32(-65536)
    return plsc.bitcast(r, jnp.float32)


def _body(logits_hbm, vf_hbm, out_hbm, table_v, lbuf, qring, pbring, obuf):
    cid = lax.axis_index("c")
    sid = lax.axis_index("s")
    wid = sid * 2 + cid
    b = wid // WORKERS_PER_IMG
    r0 = (wid % WORKERS_PER_IMG) * ROWS_PER_WORKER

    pltpu.sync_copy(vf_hbm, table_v)

    zero16 = jnp.zeros((LANES,), jnp.float32)

    def _zinit(i, carry):
        qring[pl.ds(i * LANES, LANES)] = zero16
        return carry

    lax.fori_loop(0, 5 * QS // LANES, _zinit, 0)

    def compute_prob_row(h):
        qslot = lax.rem(h, 4)
        pslot = lax.rem(h, 2)
        pltpu.sync_copy(logits_hbm.at[b, :, h, :], lbuf)

        def chunk(k, carry):
            o = k * LANES
            xs = [lbuf[c, pl.ds(o, LANES)] for c in range(NUM_CLASSES)]
            m = xs[0]
            for c in range(1, NUM_CLASSES):
                m = jnp.maximum(m, xs[c])
            es = []
            s = zero16
            for c in range(NUM_CLASSES):
                e = jnp.exp(xs[c] - m)
                es.append(e)
                s = s + e
            r = 1.0 / s
            for c in range(NUM_CLASSES):
                p = es[c] * r
                qring[pl.ds(qslot * QS + c * VROW + LANES + o, LANES)] = (
                    _round_bf16(p) * _WHI)
                pbring[pslot, c, pl.ds(o, LANES)] = jnp.minimum(
                    (p / _W15).astype(jnp.int32), NUM_PROB_BINS - 1)
            return carry

        lax.fori_loop(0, CHUNKS, chunk, 0)

    @pl.when(r0 > 0)
    def _():
        compute_prob_row(r0 - 1)

    compute_prob_row(r0)

    def row_body(h, carry):
        @pl.when(h < H - 1)
        def _():
            compute_prob_row(h + 1)

        s0 = lax.rem(h, 4)
        sm1 = jnp.where(h == 0, 4, lax.rem(h - 1, 4))
        sp1 = jnp.where(h < H - 1, lax.rem(h + 1, 4), 4)
        pslot = lax.rem(h, 2)

        def hchunk(k, carry):
            o = k * LANES
            lanes = lax.iota(jnp.int32, LANES)
            lm = lanes - 1
            lp = lanes + 1
            cals = []
            s = zero16
            for c in range(NUM_CLASSES):
                col = c * VROW + LANES + o
                bm1 = sm1 * QS + col
                b0 = s0 * QS + col
                bp1 = sp1 * QS + col
                t0 = plsc.load_gather(qring, [lm + bm1])
                t1 = qring[pl.ds(bm1, LANES)]
                t2 = plsc.load_gather(qring, [lp + bm1])
                t3 = plsc.load_gather(qring, [lm + b0])
                t4 = qring[pl.ds(b0, LANES)]
                t5 = plsc.load_gather(qring, [lp + b0])
                t6 = plsc.load_gather(qring, [lm + bp1])
                t7 = qring[pl.ds(bp1, LANES)]
                t8 = plsc.load_gather(qring, [lp + bp1])
                hs = (((t0 + t1) + (t2 + t3))
                      + ((t4 + t5) + (t6 + t7))) + t8
                lb = jnp.minimum((hs / _W9).astype(jnp.int32), NUM_NEIGH - 1)
                pb = pbring[pslot, c, pl.ds(o, LANES)]
                idx = lb * NUM_PROB_BINS + pb + c * (NUM_NEIGH * NUM_PROB_BINS)
                cal = plsc.load_gather(table_v, [idx])
                cals.append(cal)
                s = s + cal
            s = jnp.where(s == 0.0, 1.0, s)
            r = 1.0 / s
            for c in range(NUM_CLASSES):
                obuf[c, pl.ds(o, LANES)] = cals[c] * r
            return carry

        lax.fori_loop(0, CHUNKS, hchunk, 0)

        pltpu.sync_copy(obuf, out_hbm.at[b, :, h, :])
        return carry

    lax.fori_loop(r0, r0 + ROWS_PER_WORKER, row_body, 0)


def kernel(logits, val_freqs):
    vf_flat = jnp.pad(val_freqs.reshape(-1), (0, TABLE_PAD - TABLE_N))
    mesh = plsc.VectorSubcoreMesh(core_axis_name="c", subcore_axis_name="s")
    k = functools.partial(
        pl.kernel,
        mesh=mesh,
        out_type=jax.ShapeDtypeStruct((B, NUM_CLASSES, H, W), jnp.float32),
        scratch_types=[
            pltpu.VMEM((TABLE_PAD,), jnp.float32),
            pltpu.VMEM((NUM_CLASSES, W), jnp.float32),
            pltpu.VMEM((5 * QS,), jnp.float32),
            pltpu.VMEM((2, NUM_CLASSES, W), jnp.int32),
            pltpu.VMEM((NUM_CLASSES, W), jnp.float32),
        ],
        compiler_params=pltpu.CompilerParams(needs_layout_passes=False),
    )(_body)
    return k(logits, vf_flat)


# speedup vs baseline: 455.6583x; 1.1805x over previous
import functools

import jax
import jax.numpy as jnp
import numpy as np
from jax import lax
from jax.experimental import pallas as pl
from jax.experimental.pallas import tpu as pltpu
from jax.experimental.pallas import tpu_sc as plsc

NUM_PROB_BINS = 15
NUM_CLASSES = 19
NUM_NEIGH = 9
B, H, W = 8, 512, 512
LANES = 16
CHUNKS = W // LANES
NUM_WORKERS = 32
WORKERS_PER_IMG = NUM_WORKERS // B
ROWS_PER_WORKER = H // WORKERS_PER_IMG
TABLE_N = NUM_CLASSES * NUM_NEIGH * NUM_PROB_BINS
TABLE_PAD = 2576
VROW = 544
QS = NUM_CLASSES * VROW

_WHI = np.float32(
    np.asarray(np.float32(1.0 / 9.0)).astype(__import__("ml_dtypes").bfloat16)
)
_W9 = np.float32(1.0) / np.float32(NUM_NEIGH)
_W15 = np.float32(1.0) / np.float32(NUM_PROB_BINS)


def _round_bf16(x):
    u = plsc.bitcast(x, jnp.int32)
    lsb = lax.shift_right_logical(u, 16) & 1
    r = (u + 32767 + lsb) & jnp.int32(-65536)
    return plsc.bitcast(r, jnp.float32)


def _body(logits_hbm, vf_hbm, out_hbm, table_v, lbuf, qring, pbring, obuf,
          ise, iso, ose, oso):
    cid = lax.axis_index("c")
    sid = lax.axis_index("s")
    wid = sid * 2 + cid
    b = wid // WORKERS_PER_IMG
    r0 = (wid % WORKERS_PER_IMG) * ROWS_PER_WORKER

    pltpu.sync_copy(vf_hbm, table_v)

    zero16 = jnp.zeros((LANES,), jnp.float32)

    def _zinit(i, carry):
        qring[pl.ds(i * LANES, LANES)] = zero16
        return carry

    lax.fori_loop(0, 5 * QS // LANES, _zinit, 0)

    def softmax_row(h, ls):
        qslot = lax.rem(h, 4)
        pslot = lax.rem(h, 2)

        def chunk(k, carry):
            o = k * LANES
            xs = [lbuf[ls, c, pl.ds(o, LANES)] for c in range(NUM_CLASSES)]
            m = xs[0]
            for c in range(1, NUM_CLASSES):
                m = jnp.maximum(m, xs[c])
            es = []
            s = zero16
            for c in range(NUM_CLASSES):
                e = jnp.exp(xs[c] - m)
                es.append(e)
                s = s + e
            r = 1.0 / s
            for c in range(NUM_CLASSES):
                p = es[c] * r
                qring[pl.ds(qslot * QS + c * VROW + LANES + o, LANES)] = (
                    _round_bf16(p) * _WHI)
                pbring[pslot, c, pl.ds(o, LANES)] = jnp.minimum(
                    (p / _W15).astype(jnp.int32), NUM_PROB_BINS - 1)
            return carry

        lax.fori_loop(0, CHUNKS, chunk, 0)

    def compute_out_row(h, oslot):
        s0 = lax.rem(h, 4)
        sm1 = jnp.where(h == 0, 4, lax.rem(h - 1, 4))
        sp1 = jnp.where(h < H - 1, lax.rem(h + 1, 4), 4)
        pslot = lax.rem(h, 2)

        def hchunk(k, carry):
            o = k * LANES
            lanes = lax.iota(jnp.int32, LANES)
            lm = lanes - 1
            lp = lanes + 1
            cals = []
            s = zero16
            for c in range(NUM_CLASSES):
                col = c * VROW + LANES + o
                bm1 = sm1 * QS + col
                b0 = s0 * QS + col
                bp1 = sp1 * QS + col
                t0 = plsc.load_gather(qring, [lm + bm1])
                t1 = qring[pl.ds(bm1, LANES)]
                t2 = plsc.load_gather(qring, [lp + bm1])
                t3 = plsc.load_gather(qring, [lm + b0])
                t4 = qring[pl.ds(b0, LANES)]
                t5 = plsc.load_gather(qring, [lp + b0])
                t6 = plsc.load_gather(qring, [lm + bp1])
                t7 = qring[pl.ds(bp1, LANES)]
                t8 = plsc.load_gather(qring, [lp + bp1])
                hs = (((t0 + t1) + (t2 + t3))
                      + ((t4 + t5) + (t6 + t7))) + t8
                lb = jnp.minimum((hs / _W9).astype(jnp.int32), NUM_NEIGH - 1)
                pb = pbring[pslot, c, pl.ds(o, LANES)]
                idx = lb * NUM_PROB_BINS + pb + c * (NUM_NEIGH * NUM_PROB_BINS)
                cal = plsc.load_gather(table_v, [idx])
                cals.append(cal)
                s = s + cal
            s = jnp.where(s == 0.0, 1.0, s)
            r = 1.0 / s
            for c in range(NUM_CLASSES):
                obuf[oslot, c, pl.ds(o, LANES)] = cals[c] * r
            return carry

        lax.fori_loop(0, CHUNKS, hchunk, 0)

    def start_in(h, ls, sem):
        pltpu.async_copy(logits_hbm.at[b, :, h, :], lbuf.at[ls], sem)

    def wait_in(h, ls, sem):
        pltpu.make_async_copy(
            logits_hbm.at[b, :, h, :], lbuf.at[ls], sem).wait()

    def start_out(h, os, sem):
        pltpu.async_copy(obuf.at[os], out_hbm.at[b, :, h, :], sem)

    def wait_out(h, os, sem):
        pltpu.make_async_copy(
            obuf.at[os], out_hbm.at[b, :, h, :], sem).wait()

    @pl.when(r0 > 0)
    def _():
        pltpu.sync_copy(logits_hbm.at[b, :, r0 - 1, :], lbuf.at[0])
        softmax_row(r0 - 1, 0)

    pltpu.sync_copy(logits_hbm.at[b, :, r0, :], lbuf.at[0])
    softmax_row(r0, 0)
    start_in(r0 + 1, 1, iso)

    def row_pair(i, carry):
        h = r0 + 2 * i

        wait_in(h + 1, 1, iso)

        @pl.when(h + 2 < H)
        def _():
            start_in(h + 2, 0, ise)

        softmax_row(h + 1, 1)

        @pl.when(i > 0)
        def _():
            wait_out(h - 2, 0, ose)

        compute_out_row(h, 0)
        start_out(h, 0, ose)

        @pl.when(h + 2 < H)
        def _():
            wait_in(h + 2, 0, ise)
            softmax_row(h + 2, 0)

        @pl.when(h + 3 < r0 + ROWS_PER_WORKER)
        def _():
            start_in(h + 3, 1, iso)

        @pl.when(i > 0)
        def _():
            wait_out(h - 1, 1, oso)

        compute_out_row(h + 1, 1)
        start_out(h + 1, 1, oso)
        return carry

    lax.fori_loop(0, ROWS_PER_WORKER // 2, row_pair, 0)
    wait_out(r0 + ROWS_PER_WORKER - 2, 0, ose)
    wait_out(r0 + ROWS_PER_WORKER - 1, 1, oso)


def kernel(logits, val_freqs):
    vf_flat = jnp.pad(val_freqs.reshape(-1), (0, TABLE_PAD - TABLE_N))
    mesh = plsc.VectorSubcoreMesh(core_axis_name="c", subcore_axis_name="s")
    k = functools.partial(
        pl.kernel,
        mesh=mesh,
        out_type=jax.ShapeDtypeStruct((B, NUM_CLASSES, H, W), jnp.float32),
        scratch_types=[
            pltpu.VMEM((TABLE_PAD,), jnp.float32),
            pltpu.VMEM((2, NUM_CLASSES, W), jnp.float32),
            pltpu.VMEM((5 * QS,), jnp.float32),
            pltpu.VMEM((2, NUM_CLASSES, W), jnp.int32),
            pltpu.VMEM((2, NUM_CLASSES, W), jnp.float32),
            pltpu.SemaphoreType.DMA,
            pltpu.SemaphoreType.DMA,
            pltpu.SemaphoreType.DMA,
            pltpu.SemaphoreType.DMA,
        ],
        compiler_params=pltpu.CompilerParams(needs_layout_passes=False),
    )(_body)
    return k(logits, vf_flat)
